# baseline (speedup 1.0000x reference)
import jax
import jax.numpy as jnp
from jax.experimental import pallas as pl
from jax.experimental.pallas import tpu as pltpu
from jax.experimental.pallas import tpu_sc as plsc

_LP = 256
_GATHER_WINDOW = 128
_TILE = 4096


def _sc_gather(emb, idx2d):
    D = emb.shape[1]
    mesh = plsc.VectorSubcoreMesh(core_axis_name="c", subcore_axis_name="s")

    @pl.kernel(out_type=jax.ShapeDtypeStruct((_LP, D), emb.dtype), mesh=mesh)
    def gather_kernel(emb_hbm, idx_hbm, out_hbm):
        def body(i_vmem, o_vmem):
            pltpu.sync_copy(emb_hbm.at[i_vmem.at[0]], o_vmem)

        pltpu.emit_pipeline(
            body,
            grid=(_LP // _GATHER_WINDOW,),
            in_specs=[pl.BlockSpec((1, _GATHER_WINDOW), lambda i: (0, i))],
            out_specs=[pl.BlockSpec((_GATHER_WINDOW, D), lambda i: (i, 0))],
            core_axis_name="s",
            dimension_semantics=(pltpu.PARALLEL,),
        )(idx_hbm, out_hbm)

    return gather_kernel(emb, idx2d)


def _mlp_logits_lse(gathered, L, W1, b1r, W2, b2r):
    LP, D = gathered.shape
    H = W1.shape[1]
    V = W2.shape[1]
    T = _TILE
    nt = pl.cdiv(V, T)

    def kfn(g_ref, w1_ref, b1_ref, w2_ref, b2_ref, out_ref, lse_ref,
            h_ref, m_ref, s_ref):
        j = pl.program_id(0)

        @pl.when(j == 0)
        def _():
            lane = jax.lax.broadcasted_iota(jnp.int32, (1, LP), 1)
            maskr = (lane < L).astype(jnp.float32)
            embr = jnp.dot(maskr, g_ref[...],
                           preferred_element_type=jnp.float32)
            hr = jnp.dot(embr, w1_ref[...],
                         preferred_element_type=jnp.float32) + b1_ref[...]
            h_ref[...] = jnp.maximum(hr, 0.0)
            m_ref[...] = jnp.full((1, 1), -jnp.inf, jnp.float32)
            s_ref[...] = jnp.zeros((1, 1), jnp.float32)

        t = jnp.dot(h_ref[...], w2_ref[...],
                    preferred_element_type=jnp.float32) + b2_ref[...]
        col = j * T + jax.lax.broadcasted_iota(jnp.int32, (1, T), 1)
        t = jnp.where(col < V, t, -jnp.inf)
        out_ref[...] = t

        m_old = m_ref[...]
        tmax = jnp.max(t, axis=1, keepdims=True)
        m_new = jnp.maximum(m_old, tmax)
        s_ref[...] = (s_ref[...] * jnp.exp(m_old - m_new)
                      + jnp.sum(jnp.exp(t - m_new), axis=1, keepdims=True))
        m_ref[...] = m_new

        @pl.when(j == nt - 1)
        def _():
            lse_ref[...] = m_ref[...] + jnp.log(s_ref[...])

    return pl.pallas_call(
        kfn,
        grid=(nt,),
        in_specs=[
            pl.BlockSpec((LP, D), lambda j: (0, 0)),
            pl.BlockSpec((D, H), lambda j: (0, 0)),
            pl.BlockSpec((1, H), lambda j: (0, 0)),
            pl.BlockSpec((H, T), lambda j: (0, j)),
            pl.BlockSpec((1, T), lambda j: (0, j)),
        ],
        out_specs=[
            pl.BlockSpec((1, T), lambda j: (0, j)),
            pl.BlockSpec((1, 1), lambda j: (0, 0)),
        ],
        out_shape=[
            jax.ShapeDtypeStruct((1, V), jnp.float32),
            jax.ShapeDtypeStruct((1, 1), jnp.float32),
        ],
        scratch_shapes=[
            pltpu.VMEM((1, H), jnp.float32),
            pltpu.VMEM((1, 1), jnp.float32),
            pltpu.VMEM((1, 1), jnp.float32),
        ],
    )(gathered, W1, b1r, W2, b2r)


def _subtract_lse(logits, lse):
    V = logits.shape[1]
    T = _TILE
    nt = pl.cdiv(V, T)

    def kfn(l_ref, lse_ref, o_ref):
        o_ref[...] = l_ref[...] - lse_ref[...]

    return pl.pallas_call(
        kfn,
        grid=(nt,),
        in_specs=[
            pl.BlockSpec((1, T), lambda j: (0, j)),
            pl.BlockSpec((1, 1), lambda j: (0, 0)),
        ],
        out_specs=pl.BlockSpec((1, T), lambda j: (0, j)),
        out_shape=jax.ShapeDtypeStruct((1, V), jnp.float32),
        input_output_aliases={0: 0},
    )(logits, lse)


def kernel(inputs, emb, W1, b1, W2, b2):
    L = inputs.shape[0]
    H = W1.shape[1]
    V = W2.shape[1]
    idx = jnp.zeros((_LP,), jnp.int32).at[:L].set(inputs.astype(jnp.int32))
    gathered = _sc_gather(emb, idx.reshape(1, _LP))
    logits, lse = _mlp_logits_lse(gathered, L, W1, b1.reshape(1, H),
                                  W2, b2.reshape(1, V))
    return _subtract_lse(logits, lse)
